# baseline (speedup 1.0000x reference)
import functools

import jax
import jax.numpy as jnp
from jax import lax
from jax.experimental import pallas as pl
from jax.experimental.pallas import tpu as pltpu
from jax.experimental.pallas import tpu_sc as plsc

N = 10000
D = 128
E = 320000
NC = 2
NS = 16
NWORK = NC * NS
B = 128
EPW = 10240
NWIN = EPW // B
EPAD = NWORK * EPW
NPAD = N + 112
RPT = NPAD // NS
NBUF = 2
CHUNK = 16
NCHUNK = NWIN // CHUNK
_SLAB_CHUNKS = [(0, 128), (128, 128), (256, 128), (384, 128), (512, 120)]


_SEGSUM_CACHE = {}


def _segsum(table, gidx, sidx, zeros, width):
    if width in _SEGSUM_CACHE:
        return _SEGSUM_CACHE[width](table, gidx, sidx, zeros)
    mesh = plsc.VectorSubcoreMesh(core_axis_name="c", subcore_axis_name="s")

    @functools.partial(
        pl.kernel,
        out_type=jax.ShapeDtypeStruct((NC * NPAD, width), jnp.float32),
        mesh=mesh,
        compiler_params=pltpu.CompilerParams(use_tc_tiling_on_sc=(width == D)),
        scratch_types=[
            pltpu.VMEM_SHARED((NPAD, width), jnp.float32),
            *[pltpu.SemaphoreType.DMA for _ in range(2 * NBUF)],
        ],
    )
    def body(table_ref, g_ref, s_ref, z_ref, out_ref, acc, *sems):
        pl.run_scoped(
            functools.partial(_inner, table_ref, g_ref, s_ref, z_ref,
                              out_ref, acc, sems),
            pltpu.VMEM((CHUNK, B), jnp.int32),
            pltpu.VMEM((CHUNK, B), jnp.int32),
            *[pltpu.VMEM((B, width), jnp.float32) for _ in range(NBUF)],
        )

    def _inner(table_ref, g_ref, s_ref, z_ref, out_ref, acc, sems,
               gi, si, *bufs):
        gsems = sems[:NBUF]
        ssems = sems[NBUF:]
        c = lax.axis_index("c")
        s = lax.axis_index("s")
        wid = s * NC + c
        for off, sz in _SLAB_CHUNKS:
            pltpu.sync_copy(z_ref.at[pl.ds(s * RPT + off, sz)],
                            acc.at[pl.ds(s * RPT + off, sz)])
        plsc.subcore_barrier()

        def fire_g(j, w):
            pltpu.async_copy(table_ref.at[gi.at[w]], bufs[j], gsems[j])

        def wait_g(j, w):
            pltpu.make_async_copy(z_ref.at[pl.ds(0, B)], bufs[j], gsems[j]).wait()

        def scatter(j, w):
            pltpu.sync_copy(bufs[j], acc.at[si.at[w]], add=True)

        def chunk_(kc, carry):
            pltpu.sync_copy(g_ref.at[wid * NCHUNK + kc], gi)
            pltpu.sync_copy(s_ref.at[wid * NCHUNK + kc], si)

            def round_(k, carry2):
                for j in range(NBUF):
                    fire_g(j, k * NBUF + j)
                for j in range(NBUF):
                    w = k * NBUF + j
                    wait_g(j, w)
                    scatter(j, w)
                return carry2

            lax.fori_loop(0, CHUNK // NBUF, round_, 0)
            return carry

        lax.fori_loop(0, NCHUNK, chunk_, 0)
        plsc.subcore_barrier()
        for off, sz in _SLAB_CHUNKS:
            pltpu.sync_copy(acc.at[pl.ds(s * RPT + off, sz)],
                            out_ref.at[pl.ds(c * NPAD + s * RPT + off, sz)])

    _SEGSUM_CACHE[width] = body
    return body(table, gidx, sidx, zeros)


BR = 1000


def _tc1_body(x_ref, w_ref, d0_ref, d1_ref, g_ref, dinv_ref):
    deg = 1.0 + d0_ref[...] + d1_ref[...]
    dinv = lax.rsqrt(deg)
    dinv_ref[...] = dinv
    g_ref[...] = dinv[:, :1] * jnp.dot(
        x_ref[...], w_ref[...], preferred_element_type=jnp.float32)


def _tc1(x, w1, d0, d1):
    return pl.pallas_call(
        _tc1_body,
        grid=(N // BR,),
        in_specs=[
            pl.BlockSpec((BR, D), lambda i: (i, 0)),
            pl.BlockSpec((D, D), lambda i: (0, 0)),
            pl.BlockSpec((BR, 16), lambda i: (i, 0)),
            pl.BlockSpec((BR, 16), lambda i: (i, 0)),
        ],
        out_specs=[
            pl.BlockSpec((BR, D), lambda i: (i, 0)),
            pl.BlockSpec((BR, 16), lambda i: (i, 0)),
        ],
        out_shape=[
            jax.ShapeDtypeStruct((N, D), jnp.float32),
            jax.ShapeDtypeStruct((N, 16), jnp.float32),
        ],
    )(x, w1, d0, d1)


def _tc2_body(s0_ref, s1_ref, g_ref, dv_ref, b_ref, w_ref, out_ref):
    dcol = dv_ref[:, :1]
    h = jnp.maximum(dcol * (s0_ref[...] + s1_ref[...] + g_ref[...]) + b_ref[...], 0.0)
    out_ref[...] = dcol * jnp.dot(h, w_ref[...], preferred_element_type=jnp.float32)


def _tc2(s0, s1, g1, dinv16, b1, w2):
    return pl.pallas_call(
        _tc2_body,
        grid=(N // BR,),
        in_specs=[
            pl.BlockSpec((BR, D), lambda i: (i, 0)),
            pl.BlockSpec((BR, D), lambda i: (i, 0)),
            pl.BlockSpec((BR, D), lambda i: (i, 0)),
            pl.BlockSpec((BR, 16), lambda i: (i, 0)),
            pl.BlockSpec((1, D), lambda i: (0, 0)),
            pl.BlockSpec((D, D), lambda i: (0, 0)),
        ],
        out_specs=pl.BlockSpec((BR, D), lambda i: (i, 0)),
        out_shape=jax.ShapeDtypeStruct((N, D), jnp.float32),
    )(s0, s1, g1, dinv16, b1, w2)


def _tc3_body(s0_ref, s1_ref, g_ref, dv_ref, c0_ref, c1_ref, b2_ref, w3_ref,
              b3_ref, out_ref, acc_ref):
    i = pl.program_id(0)

    @pl.when(i == 0)
    def _():
        acc_ref[...] = jnp.zeros_like(acc_ref)

    dv = dv_ref[...]
    dcol = dv[:, :1]
    h2 = jnp.maximum(dcol * (s0_ref[...] + s1_ref[...] + g_ref[...]) + b2_ref[...], 0.0)
    w16 = dv * (dv + c0_ref[...] + c1_ref[...])
    acc_ref[...] += jnp.sum(w16[:, :1] * h2, axis=0, keepdims=True)

    @pl.when(i == pl.num_programs(0) - 1)
    def _():
        out_ref[...] = jnp.dot(
            acc_ref[...], w3_ref[...], preferred_element_type=jnp.float32
        ) * (1.0 / N) + b3_ref[...]


def _tc3(s0, s1, g2, dinv16, c0, c1, b2, w3, b3):
    return pl.pallas_call(
        _tc3_body,
        grid=(N // BR,),
        in_specs=[
            pl.BlockSpec((BR, D), lambda i: (i, 0)),
            pl.BlockSpec((BR, D), lambda i: (i, 0)),
            pl.BlockSpec((BR, D), lambda i: (i, 0)),
            pl.BlockSpec((BR, 16), lambda i: (i, 0)),
            pl.BlockSpec((BR, 16), lambda i: (i, 0)),
            pl.BlockSpec((BR, 16), lambda i: (i, 0)),
            pl.BlockSpec((1, D), lambda i: (0, 0)),
            pl.BlockSpec((D, D), lambda i: (0, 0)),
            pl.BlockSpec((1, D), lambda i: (0, 0)),
        ],
        out_specs=pl.BlockSpec((1, D), lambda i: (0, 0)),
        out_shape=jax.ShapeDtypeStruct((1, D), jnp.float32),
        scratch_shapes=[pltpu.VMEM((1, D), jnp.float32)],
    )(s0, s1, g2, dinv16, c0, c1, b2, w3, b3)


def kernel(x, edge_index, W1, b1, W2, b2, W3, b3):
    ei = edge_index.astype(jnp.int32)
    src, dst = ei[0], ei[1]
    pad = EPAD - E
    padidx = jnp.arange(pad, dtype=jnp.int32)
    zpad = jnp.zeros((pad,), jnp.int32)
    trash = N + (padidx % 16)
    src_g = jnp.concatenate([src, zpad]).reshape(NWORK * NCHUNK, CHUNK, B)
    dst_s = jnp.concatenate([dst, trash]).reshape(NWORK * NCHUNK, CHUNK, B)
    dst_g = jnp.concatenate([dst, zpad]).reshape(NWORK * NCHUNK, CHUNK, B)
    src_s = jnp.concatenate([src, trash]).reshape(NWORK * NCHUNK, CHUNK, B)

    zeros128 = jnp.zeros((NPAD, D), jnp.float32)
    zeros16 = jnp.zeros((NPAD, 16), jnp.float32)
    ones16 = jnp.ones((N, 16), jnp.float32)

    degp = _segsum(ones16, src_g, dst_s, zeros16, 16)
    g1, dinv16 = _tc1(x, W1, degp[:N], degp[NPAD:NPAD + N])

    cp = _segsum(dinv16, dst_g, src_s, zeros16, 16)

    s1p = _segsum(g1, src_g, dst_s, zeros128, D)
    g2 = _tc2(s1p[:N], s1p[NPAD:NPAD + N], g1, dinv16, b1.reshape(1, D), W2)

    s2p = _segsum(g2, src_g, dst_s, zeros128, D)
    out = _tc3(s2p[:N], s2p[NPAD:NPAD + N], g2, dinv16,
               cp[:N], cp[NPAD:NPAD + N],
               b2.reshape(1, D), W3, b3.reshape(1, D))
    return out[0]


# speedup vs baseline: 2.7541x; 2.7541x over previous
import functools

import jax
import jax.numpy as jnp
from jax import lax
from jax.experimental import pallas as pl
from jax.experimental.pallas import tpu as pltpu
from jax.experimental.pallas import tpu_sc as plsc

N = 10000
D = 128
E = 320000
NC = 2
NS = 16
NWORK = NC * NS
B = 128
EPW = 10240
NWIN = EPW // B
EPAD = NWORK * EPW
NPAD = N + 112
RPT = NPAD // NS
NBUF = 2
CHUNK = 16
NCHUNK = NWIN // CHUNK
_SLAB_CHUNKS = [(0, 128), (128, 128), (256, 128), (384, 128), (512, 120)]


_SEGSUM_CACHE = {}


def _segsum(table, gidx, sidx, zeros, width):
    if width in _SEGSUM_CACHE:
        return _SEGSUM_CACHE[width](table, gidx, sidx, zeros)
    mesh = plsc.VectorSubcoreMesh(core_axis_name="c", subcore_axis_name="s")

    @functools.partial(
        pl.kernel,
        out_type=jax.ShapeDtypeStruct((NC * NPAD, width), jnp.float32),
        mesh=mesh,
        compiler_params=pltpu.CompilerParams(use_tc_tiling_on_sc=(width == D)),
        scratch_types=[
            pltpu.VMEM_SHARED((NPAD, width), jnp.float32),
            *[pltpu.SemaphoreType.DMA for _ in range(2 * NBUF)],
        ],
    )
    def body(table_ref, g_ref, s_ref, z_ref, out_ref, acc, *sems):
        pl.run_scoped(
            functools.partial(_inner, table_ref, g_ref, s_ref, z_ref,
                              out_ref, acc, sems),
            pltpu.VMEM((CHUNK, B), jnp.int32),
            pltpu.VMEM((CHUNK, B), jnp.int32),
            *[pltpu.VMEM((B, width), jnp.float32) for _ in range(NBUF)],
        )

    def _inner(table_ref, g_ref, s_ref, z_ref, out_ref, acc, sems,
               gi, si, *bufs):
        gsems = sems[:NBUF]
        ssems = sems[NBUF:]
        c = lax.axis_index("c")
        s = lax.axis_index("s")
        wid = s * NC + c
        for off, sz in _SLAB_CHUNKS:
            pltpu.sync_copy(z_ref.at[pl.ds(s * RPT + off, sz)],
                            acc.at[pl.ds(s * RPT + off, sz)])
        plsc.subcore_barrier()

        def fire_g(j, w):
            pltpu.async_copy(table_ref.at[gi.at[w]], bufs[j], gsems[j])

        def wait_g(j, w):
            pltpu.make_async_copy(z_ref.at[pl.ds(0, B)], bufs[j], gsems[j]).wait()

        def scatter(j, w):
            pltpu.sync_copy(bufs[j], acc.at[si.at[w]], add=True)

        def chunk_(kc, carry):
            pltpu.sync_copy(g_ref.at[wid * NCHUNK + kc], gi)
            pltpu.sync_copy(s_ref.at[wid * NCHUNK + kc], si)

            def round_(k, carry2):
                for j in range(NBUF):
                    fire_g(j, k * NBUF + j)
                for j in range(NBUF):
                    w = k * NBUF + j
                    wait_g(j, w)
                    scatter(j, w)
                return carry2

            lax.fori_loop(0, CHUNK // NBUF, round_, 0)
            return carry

        lax.fori_loop(0, NCHUNK, chunk_, 0)
        plsc.subcore_barrier()
        for off, sz in _SLAB_CHUNKS:
            pltpu.sync_copy(acc.at[pl.ds(s * RPT + off, sz)],
                            out_ref.at[pl.ds(c * NPAD + s * RPT + off, sz)])

    _SEGSUM_CACHE[width] = body
    return body(table, gidx, sidx, zeros)


BR = 1000


def _tc1_body(x_ref, w_ref, d0_ref, d1_ref, g_ref, dinv_ref):
    deg = 1.0 + d0_ref[...] + d1_ref[...]
    dinv = lax.rsqrt(deg)
    dinv_ref[...] = dinv
    g_ref[...] = dinv[:, :1] * jnp.dot(
        x_ref[...], w_ref[...], preferred_element_type=jnp.float32)


def _tc1(x, w1, d0, d1):
    return pl.pallas_call(
        _tc1_body,
        grid=(N // BR,),
        in_specs=[
            pl.BlockSpec((BR, D), lambda i: (i, 0)),
            pl.BlockSpec((D, D), lambda i: (0, 0)),
            pl.BlockSpec((BR, 16), lambda i: (i, 0)),
            pl.BlockSpec((BR, 16), lambda i: (i, 0)),
        ],
        out_specs=[
            pl.BlockSpec((BR, D), lambda i: (i, 0)),
            pl.BlockSpec((BR, 16), lambda i: (i, 0)),
        ],
        out_shape=[
            jax.ShapeDtypeStruct((N, D), jnp.float32),
            jax.ShapeDtypeStruct((N, 16), jnp.float32),
        ],
    )(x, w1, d0, d1)


def _tc2_body(s0_ref, s1_ref, g_ref, dv_ref, b_ref, w_ref, out_ref):
    dcol = dv_ref[:, :1]
    h = jnp.maximum(dcol * (s0_ref[...] + s1_ref[...] + g_ref[...]) + b_ref[...], 0.0)
    out_ref[...] = dcol * jnp.dot(h, w_ref[...], preferred_element_type=jnp.float32)


def _tc2(s0, s1, g1, dinv16, b1, w2):
    return pl.pallas_call(
        _tc2_body,
        grid=(N // BR,),
        in_specs=[
            pl.BlockSpec((BR, D), lambda i: (i, 0)),
            pl.BlockSpec((BR, D), lambda i: (i, 0)),
            pl.BlockSpec((BR, D), lambda i: (i, 0)),
            pl.BlockSpec((BR, 16), lambda i: (i, 0)),
            pl.BlockSpec((1, D), lambda i: (0, 0)),
            pl.BlockSpec((D, D), lambda i: (0, 0)),
        ],
        out_specs=pl.BlockSpec((BR, D), lambda i: (i, 0)),
        out_shape=jax.ShapeDtypeStruct((N, D), jnp.float32),
    )(s0, s1, g1, dinv16, b1, w2)


def _tc3_body(s0_ref, s1_ref, g_ref, dv_ref, c0_ref, c1_ref, b2_ref, w3_ref,
              b3_ref, out_ref, acc_ref):
    i = pl.program_id(0)

    @pl.when(i == 0)
    def _():
        acc_ref[...] = jnp.zeros_like(acc_ref)

    dv = dv_ref[...]
    dcol = dv[:, :1]
    h2 = jnp.maximum(dcol * (s0_ref[...] + s1_ref[...] + g_ref[...]) + b2_ref[...], 0.0)
    w16 = dv * (dv + c0_ref[...] + c1_ref[...])
    acc_ref[...] += jnp.sum(w16[:, :1] * h2, axis=0, keepdims=True)

    @pl.when(i == pl.num_programs(0) - 1)
    def _():
        out_ref[...] = jnp.dot(
            acc_ref[...], w3_ref[...], preferred_element_type=jnp.float32
        ) * (1.0 / N) + b3_ref[...]


def _tc3(s0, s1, g2, dinv16, c0, c1, b2, w3, b3):
    return pl.pallas_call(
        _tc3_body,
        grid=(N // BR,),
        in_specs=[
            pl.BlockSpec((BR, D), lambda i: (i, 0)),
            pl.BlockSpec((BR, D), lambda i: (i, 0)),
            pl.BlockSpec((BR, D), lambda i: (i, 0)),
            pl.BlockSpec((BR, 16), lambda i: (i, 0)),
            pl.BlockSpec((BR, 16), lambda i: (i, 0)),
            pl.BlockSpec((BR, 16), lambda i: (i, 0)),
            pl.BlockSpec((1, D), lambda i: (0, 0)),
            pl.BlockSpec((D, D), lambda i: (0, 0)),
            pl.BlockSpec((1, D), lambda i: (0, 0)),
        ],
        out_specs=pl.BlockSpec((1, D), lambda i: (0, 0)),
        out_shape=jax.ShapeDtypeStruct((1, D), jnp.float32),
        scratch_shapes=[pltpu.VMEM((1, D), jnp.float32)],
    )(s0, s1, g2, dinv16, c0, c1, b2, w3, b3)


def kernel(x, edge_index, W1, b1, W2, b2, W3, b3):
    ei = edge_index.astype(jnp.int32)
    src, dst = ei[0], ei[1]
    pad = EPAD - E
    padidx = jnp.arange(pad, dtype=jnp.int32)
    zpad = (padidx * 37) % N
    trash = N + padidx % 112
    src_g = jnp.concatenate([src, zpad]).reshape(NWORK * NCHUNK, CHUNK, B)
    dst_s = jnp.concatenate([dst, trash]).reshape(NWORK * NCHUNK, CHUNK, B)
    dst_g = jnp.concatenate([dst, zpad]).reshape(NWORK * NCHUNK, CHUNK, B)
    src_s = jnp.concatenate([src, trash]).reshape(NWORK * NCHUNK, CHUNK, B)

    zeros128 = jnp.zeros((NPAD, D), jnp.float32)
    zeros16 = jnp.zeros((NPAD, 16), jnp.float32)
    ones16 = jnp.ones((N, 16), jnp.float32)

    degp = _segsum(ones16, src_g, dst_s, zeros16, 16)
    g1, dinv16 = _tc1(x, W1, degp[:N], degp[NPAD:NPAD + N])

    cp = _segsum(dinv16, dst_g, src_s, zeros16, 16)

    s1p = _segsum(g1, src_g, dst_s, zeros128, D)
    g2 = _tc2(s1p[:N], s1p[NPAD:NPAD + N], g1, dinv16, b1.reshape(1, D), W2)

    s2p = _segsum(g2, src_g, dst_s, zeros128, D)
    out = _tc3(s2p[:N], s2p[NPAD:NPAD + N], g2, dinv16,
               cp[:N], cp[NPAD:NPAD + N],
               b2.reshape(1, D), W3, b3.reshape(1, D))
    return out[0]


# speedup vs baseline: 2.9706x; 1.0786x over previous
import functools

import jax
import jax.numpy as jnp
from jax import lax
from jax.experimental import pallas as pl
from jax.experimental.pallas import tpu as pltpu
from jax.experimental.pallas import tpu_sc as plsc

N = 10000
D = 128
E = 320000
NC = 2
NS = 16
NWORK = NC * NS
B = 128
EPW = 10240
NWIN = EPW // B
EPAD = NWORK * EPW
NPAD = N + 112
RPT = NPAD // NS
NBUF = 2
CHUNK = 16
NCHUNK = NWIN // CHUNK
_SLAB_CHUNKS = [(0, 128), (128, 128), (256, 128), (384, 128), (512, 120)]


_SEGSUM_CACHE = {}


def _segsum(table, gidx, sidx, zeros, width):
    if width in _SEGSUM_CACHE:
        return _SEGSUM_CACHE[width](table, gidx, sidx, zeros)
    mesh = plsc.VectorSubcoreMesh(core_axis_name="c", subcore_axis_name="s")

    @functools.partial(
        pl.kernel,
        out_type=jax.ShapeDtypeStruct((NC * NPAD, width), jnp.float32),
        mesh=mesh,
        compiler_params=pltpu.CompilerParams(use_tc_tiling_on_sc=(width == D)),
        scratch_types=[
            pltpu.VMEM_SHARED((NPAD, width), jnp.float32),
            *[pltpu.SemaphoreType.DMA for _ in range(2 * NBUF)],
        ],
    )
    def body(table_ref, g_ref, s_ref, z_ref, out_ref, acc, *sems):
        pl.run_scoped(
            functools.partial(_inner, table_ref, g_ref, s_ref, z_ref,
                              out_ref, acc, sems),
            pltpu.VMEM((CHUNK, B), jnp.int32),
            pltpu.VMEM((CHUNK, B), jnp.int32),
            *[pltpu.VMEM((B, width), jnp.float32) for _ in range(NBUF)],
        )

    def _inner(table_ref, g_ref, s_ref, z_ref, out_ref, acc, sems,
               gi, si, *bufs):
        gsems = sems[:NBUF]
        ssems = sems[NBUF:]
        c = lax.axis_index("c")
        s = lax.axis_index("s")
        wid = s * NC + c
        for off, sz in _SLAB_CHUNKS:
            pltpu.sync_copy(z_ref.at[pl.ds(s * RPT + off, sz)],
                            acc.at[pl.ds(s * RPT + off, sz)])
        plsc.subcore_barrier()

        def fire_g(j, w):
            pltpu.async_copy(table_ref.at[gi.at[w]], bufs[j], gsems[j])

        def wait_g(j, w):
            pltpu.make_async_copy(z_ref.at[pl.ds(0, B)], bufs[j], gsems[j]).wait()

        def scatter(j, w):
            pltpu.sync_copy(bufs[j], acc.at[si.at[w]], add=True)

        def chunk_(kc, carry):
            pltpu.sync_copy(g_ref.at[wid * NCHUNK + kc], gi)
            pltpu.sync_copy(s_ref.at[wid * NCHUNK + kc], si)

            def round_(k, carry2):
                for j in range(NBUF):
                    fire_g(j, k * NBUF + j)
                for j in range(NBUF):
                    w = k * NBUF + j
                    wait_g(j, w)
                    scatter(j, w)
                return carry2

            lax.fori_loop(0, CHUNK // NBUF, round_, 0)
            return carry

        lax.fori_loop(0, NCHUNK, chunk_, 0)
        plsc.subcore_barrier()
        for off, sz in _SLAB_CHUNKS:
            pltpu.sync_copy(acc.at[pl.ds(s * RPT + off, sz)],
                            out_ref.at[pl.ds(c * NPAD + s * RPT + off, sz)])

    _SEGSUM_CACHE[width] = body
    return body(table, gidx, sidx, zeros)


_HIST_CACHE = []
_HNB = 8


def _histones(sidx, zeros):
    if _HIST_CACHE:
        return _HIST_CACHE[0](sidx, zeros)
    mesh = plsc.VectorSubcoreMesh(core_axis_name="c", subcore_axis_name="s")

    @functools.partial(
        pl.kernel,
        out_type=jax.ShapeDtypeStruct((NC * NPAD, 16), jnp.float32),
        mesh=mesh,
        compiler_params=pltpu.CompilerParams(use_tc_tiling_on_sc=False),
        scratch_types=[
            pltpu.VMEM_SHARED((NPAD, 16), jnp.float32),
            *[pltpu.SemaphoreType.DMA for _ in range(_HNB)],
        ],
    )
    def body(s_ref, z_ref, out_ref, acc, *sems):
        pl.run_scoped(
            functools.partial(_inner, s_ref, z_ref, out_ref, acc, sems),
            pltpu.VMEM((CHUNK, B), jnp.int32),
            pltpu.VMEM((B, 16), jnp.float32),
        )

    def _inner(s_ref, z_ref, out_ref, acc, sems, si, buf):
        c = lax.axis_index("c")
        s = lax.axis_index("s")
        wid = s * NC + c
        ones = jnp.ones((16,), jnp.float32)

        def fill(i, carry):
            buf[i, pl.ds(0, 16)] = ones
            return carry

        lax.fori_loop(0, B, fill, 0)
        for off, sz in _SLAB_CHUNKS:
            pltpu.sync_copy(z_ref.at[pl.ds(s * RPT + off, sz)],
                            acc.at[pl.ds(s * RPT + off, sz)])
        plsc.subcore_barrier()

        def chunk_(kc, carry):
            pltpu.sync_copy(s_ref.at[wid * NCHUNK + kc], si)

            def round_(k, carry2):
                for j in range(_HNB):
                    w = k * _HNB + j
                    pltpu.async_copy(buf, acc.at[si.at[w]], sems[j], add=True)
                for j in range(_HNB):
                    w = k * _HNB + j
                    pltpu.make_async_copy(buf, acc.at[si.at[w]], sems[j]).wait()
                return carry2

            lax.fori_loop(0, CHUNK // _HNB, round_, 0)
            return carry

        lax.fori_loop(0, NCHUNK, chunk_, 0)
        plsc.subcore_barrier()
        for off, sz in _SLAB_CHUNKS:
            pltpu.sync_copy(acc.at[pl.ds(s * RPT + off, sz)],
                            out_ref.at[pl.ds(c * NPAD + s * RPT + off, sz)])

    _HIST_CACHE.append(body)
    return body(sidx, zeros)


BR = 1000


def _tc1_body(x_ref, w_ref, d0_ref, d1_ref, g_ref, dinv_ref):
    deg = 1.0 + d0_ref[...] + d1_ref[...]
    dinv = lax.rsqrt(deg)
    dinv_ref[...] = dinv
    g_ref[...] = dinv[:, :1] * jnp.dot(
        x_ref[...], w_ref[...], preferred_element_type=jnp.float32)


def _tc1(x, w1, d0, d1):
    return pl.pallas_call(
        _tc1_body,
        grid=(N // BR,),
        in_specs=[
            pl.BlockSpec((BR, D), lambda i: (i, 0)),
            pl.BlockSpec((D, D), lambda i: (0, 0)),
            pl.BlockSpec((BR, 16), lambda i: (i, 0)),
            pl.BlockSpec((BR, 16), lambda i: (i, 0)),
        ],
        out_specs=[
            pl.BlockSpec((BR, D), lambda i: (i, 0)),
            pl.BlockSpec((BR, 16), lambda i: (i, 0)),
        ],
        out_shape=[
            jax.ShapeDtypeStruct((N, D), jnp.float32),
            jax.ShapeDtypeStruct((N, 16), jnp.float32),
        ],
    )(x, w1, d0, d1)


def _tc2_body(s0_ref, s1_ref, g_ref, dv_ref, b_ref, w_ref, out_ref):
    dcol = dv_ref[:, :1]
    h = jnp.maximum(dcol * (s0_ref[...] + s1_ref[...] + g_ref[...]) + b_ref[...], 0.0)
    out_ref[...] = dcol * jnp.dot(h, w_ref[...], preferred_element_type=jnp.float32)


def _tc2(s0, s1, g1, dinv16, b1, w2):
    return pl.pallas_call(
        _tc2_body,
        grid=(N // BR,),
        in_specs=[
            pl.BlockSpec((BR, D), lambda i: (i, 0)),
            pl.BlockSpec((BR, D), lambda i: (i, 0)),
            pl.BlockSpec((BR, D), lambda i: (i, 0)),
            pl.BlockSpec((BR, 16), lambda i: (i, 0)),
            pl.BlockSpec((1, D), lambda i: (0, 0)),
            pl.BlockSpec((D, D), lambda i: (0, 0)),
        ],
        out_specs=pl.BlockSpec((BR, D), lambda i: (i, 0)),
        out_shape=jax.ShapeDtypeStruct((N, D), jnp.float32),
    )(s0, s1, g1, dinv16, b1, w2)


def _tc3_body(s0_ref, s1_ref, g_ref, dv_ref, c0_ref, c1_ref, b2_ref, w3_ref,
              b3_ref, out_ref, acc_ref):
    i = pl.program_id(0)

    @pl.when(i == 0)
    def _():
        acc_ref[...] = jnp.zeros_like(acc_ref)

    dv = dv_ref[...]
    dcol = dv[:, :1]
    h2 = jnp.maximum(dcol * (s0_ref[...] + s1_ref[...] + g_ref[...]) + b2_ref[...], 0.0)
    w16 = dv * (dv + c0_ref[...] + c1_ref[...])
    acc_ref[...] += jnp.sum(w16[:, :1] * h2, axis=0, keepdims=True)

    @pl.when(i == pl.num_programs(0) - 1)
    def _():
        out_ref[...] = jnp.dot(
            acc_ref[...], w3_ref[...], preferred_element_type=jnp.float32
        ) * (1.0 / N) + b3_ref[...]


def _tc3(s0, s1, g2, dinv16, c0, c1, b2, w3, b3):
    return pl.pallas_call(
        _tc3_body,
        grid=(N // BR,),
        in_specs=[
            pl.BlockSpec((BR, D), lambda i: (i, 0)),
            pl.BlockSpec((BR, D), lambda i: (i, 0)),
            pl.BlockSpec((BR, D), lambda i: (i, 0)),
            pl.BlockSpec((BR, 16), lambda i: (i, 0)),
            pl.BlockSpec((BR, 16), lambda i: (i, 0)),
            pl.BlockSpec((BR, 16), lambda i: (i, 0)),
            pl.BlockSpec((1, D), lambda i: (0, 0)),
            pl.BlockSpec((D, D), lambda i: (0, 0)),
            pl.BlockSpec((1, D), lambda i: (0, 0)),
        ],
        out_specs=pl.BlockSpec((1, D), lambda i: (0, 0)),
        out_shape=jax.ShapeDtypeStruct((1, D), jnp.float32),
        scratch_shapes=[pltpu.VMEM((1, D), jnp.float32)],
    )(s0, s1, g2, dinv16, c0, c1, b2, w3, b3)


def kernel(x, edge_index, W1, b1, W2, b2, W3, b3):
    ei = edge_index.astype(jnp.int32)
    src, dst = ei[0], ei[1]
    pad = EPAD - E
    padidx = jnp.arange(pad, dtype=jnp.int32)
    zpad = (padidx * 37) % N
    trash = N + padidx % 112
    src_g = jnp.concatenate([src, zpad]).reshape(NWORK * NCHUNK, CHUNK, B)
    dst_s = jnp.concatenate([dst, trash]).reshape(NWORK * NCHUNK, CHUNK, B)
    dst_g = jnp.concatenate([dst, zpad]).reshape(NWORK * NCHUNK, CHUNK, B)
    src_s = jnp.concatenate([src, trash]).reshape(NWORK * NCHUNK, CHUNK, B)

    zeros128 = jnp.zeros((NPAD, D), jnp.float32)
    zeros16 = jnp.zeros((NPAD, 16), jnp.float32)

    degp = _histones(dst_s, zeros16)
    g1, dinv16 = _tc1(x, W1, degp[:N], degp[NPAD:NPAD + N])

    cp = _segsum(dinv16, dst_g, src_s, zeros16, 16)

    s1p = _segsum(g1, src_g, dst_s, zeros128, D)
    g2 = _tc2(s1p[:N], s1p[NPAD:NPAD + N], g1, dinv16, b1.reshape(1, D), W2)

    s2p = _segsum(g2, src_g, dst_s, zeros128, D)
    out = _tc3(s2p[:N], s2p[NPAD:NPAD + N], g2, dinv16,
               cp[:N], cp[NPAD:NPAD + N],
               b2.reshape(1, D), W3, b3.reshape(1, D))
    return out[0]


# speedup vs baseline: 2.9768x; 1.0021x over previous
import functools

import jax
import jax.numpy as jnp
from jax import lax
from jax.experimental import pallas as pl
from jax.experimental.pallas import tpu as pltpu
from jax.experimental.pallas import tpu_sc as plsc

N = 10000
D = 128
E = 320000
NC = 2
NS = 16
NWORK = NC * NS
B = 128
EPW = 10240
NWIN = EPW // B
EPAD = NWORK * EPW
NPAD = N + 112
RPT = NPAD // NS
NBUF = 2
CHUNK = 16
NCHUNK = NWIN // CHUNK
_SLAB_CHUNKS = [(0, 128), (128, 128), (256, 128), (384, 128), (512, 120)]


_SEGSUM_CACHE = {}


def _segsum(table, gidx, sidx, zeros, width):
    if width in _SEGSUM_CACHE:
        return _SEGSUM_CACHE[width](table, gidx, sidx, zeros)
    mesh = plsc.VectorSubcoreMesh(core_axis_name="c", subcore_axis_name="s")

    @functools.partial(
        pl.kernel,
        out_type=jax.ShapeDtypeStruct((NC * NPAD, width), jnp.float32),
        mesh=mesh,
        compiler_params=pltpu.CompilerParams(use_tc_tiling_on_sc=False),
        scratch_types=[
            pltpu.VMEM_SHARED((NPAD, width), jnp.float32),
            *[pltpu.SemaphoreType.DMA for _ in range(2 * NBUF)],
        ],
    )
    def body(table_ref, g_ref, s_ref, z_ref, out_ref, acc, *sems):
        pl.run_scoped(
            functools.partial(_inner, table_ref, g_ref, s_ref, z_ref,
                              out_ref, acc, sems),
            pltpu.VMEM((CHUNK, B), jnp.int32),
            pltpu.VMEM((CHUNK, B), jnp.int32),
            *[pltpu.VMEM((B, width), jnp.float32) for _ in range(NBUF)],
        )

    def _inner(table_ref, g_ref, s_ref, z_ref, out_ref, acc, sems,
               gi, si, *bufs):
        gsems = sems[:NBUF]
        ssems = sems[NBUF:]
        c = lax.axis_index("c")
        s = lax.axis_index("s")
        wid = s * NC + c
        for off, sz in _SLAB_CHUNKS:
            pltpu.sync_copy(z_ref.at[pl.ds(s * RPT + off, sz)],
                            acc.at[pl.ds(s * RPT + off, sz)])
        plsc.subcore_barrier()

        def fire_g(j, w):
            pltpu.async_copy(table_ref.at[gi.at[w]], bufs[j], gsems[j])

        def wait_g(j, w):
            pltpu.make_async_copy(z_ref.at[pl.ds(0, B)], bufs[j], gsems[j]).wait()

        def scatter(j, w):
            pltpu.sync_copy(bufs[j], acc.at[si.at[w]], add=True)

        def chunk_(kc, carry):
            pltpu.sync_copy(g_ref.at[wid * NCHUNK + kc], gi)
            pltpu.sync_copy(s_ref.at[wid * NCHUNK + kc], si)

            def round_(k, carry2):
                for j in range(NBUF):
                    fire_g(j, k * NBUF + j)
                for j in range(NBUF):
                    w = k * NBUF + j
                    wait_g(j, w)
                    scatter(j, w)
                return carry2

            lax.fori_loop(0, CHUNK // NBUF, round_, 0)
            return carry

        lax.fori_loop(0, NCHUNK, chunk_, 0)
        plsc.subcore_barrier()
        for off, sz in _SLAB_CHUNKS:
            pltpu.sync_copy(acc.at[pl.ds(s * RPT + off, sz)],
                            out_ref.at[pl.ds(c * NPAD + s * RPT + off, sz)])

    _SEGSUM_CACHE[width] = body
    return body(table, gidx, sidx, zeros)


_HIST_CACHE = []
_HNB = 8


def _histones(sidx, zeros):
    if _HIST_CACHE:
        return _HIST_CACHE[0](sidx, zeros)
    mesh = plsc.VectorSubcoreMesh(core_axis_name="c", subcore_axis_name="s")

    @functools.partial(
        pl.kernel,
        out_type=jax.ShapeDtypeStruct((NC * NPAD, 16), jnp.float32),
        mesh=mesh,
        compiler_params=pltpu.CompilerParams(use_tc_tiling_on_sc=False),
        scratch_types=[
            pltpu.VMEM_SHARED((NPAD, 16), jnp.float32),
            *[pltpu.SemaphoreType.DMA for _ in range(_HNB)],
        ],
    )
    def body(s_ref, z_ref, out_ref, acc, *sems):
        pl.run_scoped(
            functools.partial(_inner, s_ref, z_ref, out_ref, acc, sems),
            pltpu.VMEM((CHUNK, B), jnp.int32),
            pltpu.VMEM((B, 16), jnp.float32),
        )

    def _inner(s_ref, z_ref, out_ref, acc, sems, si, buf):
        c = lax.axis_index("c")
        s = lax.axis_index("s")
        wid = s * NC + c
        ones = jnp.ones((16,), jnp.float32)

        def fill(i, carry):
            buf[i, pl.ds(0, 16)] = ones
            return carry

        lax.fori_loop(0, B, fill, 0)
        for off, sz in _SLAB_CHUNKS:
            pltpu.sync_copy(z_ref.at[pl.ds(s * RPT + off, sz)],
                            acc.at[pl.ds(s * RPT + off, sz)])
        plsc.subcore_barrier()

        def chunk_(kc, carry):
            pltpu.sync_copy(s_ref.at[wid * NCHUNK + kc], si)

            def round_(k, carry2):
                for j in range(_HNB):
                    w = k * _HNB + j
                    pltpu.async_copy(buf, acc.at[si.at[w]], sems[j], add=True)
                for j in range(_HNB):
                    w = k * _HNB + j
                    pltpu.make_async_copy(buf, acc.at[si.at[w]], sems[j]).wait()
                return carry2

            lax.fori_loop(0, CHUNK // _HNB, round_, 0)
            return carry

        lax.fori_loop(0, NCHUNK, chunk_, 0)
        plsc.subcore_barrier()
        for off, sz in _SLAB_CHUNKS:
            pltpu.sync_copy(acc.at[pl.ds(s * RPT + off, sz)],
                            out_ref.at[pl.ds(c * NPAD + s * RPT + off, sz)])

    _HIST_CACHE.append(body)
    return body(sidx, zeros)


BR = 1000


def _tc1_body(x_ref, w_ref, d0_ref, d1_ref, g_ref, dinv_ref):
    deg = 1.0 + d0_ref[...] + d1_ref[...]
    dinv = lax.rsqrt(deg)
    dinv_ref[...] = dinv
    g_ref[...] = dinv[:, :1] * jnp.dot(
        x_ref[...], w_ref[...], preferred_element_type=jnp.float32)


def _tc1(x, w1, d0, d1):
    return pl.pallas_call(
        _tc1_body,
        grid=(N // BR,),
        in_specs=[
            pl.BlockSpec((BR, D), lambda i: (i, 0)),
            pl.BlockSpec((D, D), lambda i: (0, 0)),
            pl.BlockSpec((BR, 16), lambda i: (i, 0)),
            pl.BlockSpec((BR, 16), lambda i: (i, 0)),
        ],
        out_specs=[
            pl.BlockSpec((BR, D), lambda i: (i, 0)),
            pl.BlockSpec((BR, 16), lambda i: (i, 0)),
        ],
        out_shape=[
            jax.ShapeDtypeStruct((N, D), jnp.float32),
            jax.ShapeDtypeStruct((N, 16), jnp.float32),
        ],
    )(x, w1, d0, d1)


def _tc2_body(s0_ref, s1_ref, g_ref, dv_ref, b_ref, w_ref, out_ref):
    dcol = dv_ref[:, :1]
    h = jnp.maximum(dcol * (s0_ref[...] + s1_ref[...] + g_ref[...]) + b_ref[...], 0.0)
    out_ref[...] = dcol * jnp.dot(h, w_ref[...], preferred_element_type=jnp.float32)


def _tc2(s0, s1, g1, dinv16, b1, w2):
    return pl.pallas_call(
        _tc2_body,
        grid=(N // BR,),
        in_specs=[
            pl.BlockSpec((BR, D), lambda i: (i, 0)),
            pl.BlockSpec((BR, D), lambda i: (i, 0)),
            pl.BlockSpec((BR, D), lambda i: (i, 0)),
            pl.BlockSpec((BR, 16), lambda i: (i, 0)),
            pl.BlockSpec((1, D), lambda i: (0, 0)),
            pl.BlockSpec((D, D), lambda i: (0, 0)),
        ],
        out_specs=pl.BlockSpec((BR, D), lambda i: (i, 0)),
        out_shape=jax.ShapeDtypeStruct((N, D), jnp.float32),
    )(s0, s1, g1, dinv16, b1, w2)


def _tc3_body(s0_ref, s1_ref, g_ref, dv_ref, c0_ref, c1_ref, b2_ref, w3_ref,
              b3_ref, out_ref, acc_ref):
    i = pl.program_id(0)

    @pl.when(i == 0)
    def _():
        acc_ref[...] = jnp.zeros_like(acc_ref)

    dv = dv_ref[...]
    dcol = dv[:, :1]
    h2 = jnp.maximum(dcol * (s0_ref[...] + s1_ref[...] + g_ref[...]) + b2_ref[...], 0.0)
    w16 = dv * (dv + c0_ref[...] + c1_ref[...])
    acc_ref[...] += jnp.sum(w16[:, :1] * h2, axis=0, keepdims=True)

    @pl.when(i == pl.num_programs(0) - 1)
    def _():
        out_ref[...] = jnp.dot(
            acc_ref[...], w3_ref[...], preferred_element_type=jnp.float32
        ) * (1.0 / N) + b3_ref[...]


def _tc3(s0, s1, g2, dinv16, c0, c1, b2, w3, b3):
    return pl.pallas_call(
        _tc3_body,
        grid=(N // BR,),
        in_specs=[
            pl.BlockSpec((BR, D), lambda i: (i, 0)),
            pl.BlockSpec((BR, D), lambda i: (i, 0)),
            pl.BlockSpec((BR, D), lambda i: (i, 0)),
            pl.BlockSpec((BR, 16), lambda i: (i, 0)),
            pl.BlockSpec((BR, 16), lambda i: (i, 0)),
            pl.BlockSpec((BR, 16), lambda i: (i, 0)),
            pl.BlockSpec((1, D), lambda i: (0, 0)),
            pl.BlockSpec((D, D), lambda i: (0, 0)),
            pl.BlockSpec((1, D), lambda i: (0, 0)),
        ],
        out_specs=pl.BlockSpec((1, D), lambda i: (0, 0)),
        out_shape=jax.ShapeDtypeStruct((1, D), jnp.float32),
        scratch_shapes=[pltpu.VMEM((1, D), jnp.float32)],
    )(s0, s1, g2, dinv16, c0, c1, b2, w3, b3)


def kernel(x, edge_index, W1, b1, W2, b2, W3, b3):
    ei = edge_index.astype(jnp.int32)
    src, dst = ei[0], ei[1]
    pad = EPAD - E
    padidx = jnp.arange(pad, dtype=jnp.int32)
    zpad = (padidx * 37) % N
    trash = N + padidx % 112
    src_g = jnp.concatenate([src, zpad]).reshape(NWORK * NCHUNK, CHUNK, B)
    dst_s = jnp.concatenate([dst, trash]).reshape(NWORK * NCHUNK, CHUNK, B)
    dst_g = jnp.concatenate([dst, zpad]).reshape(NWORK * NCHUNK, CHUNK, B)
    src_s = jnp.concatenate([src, trash]).reshape(NWORK * NCHUNK, CHUNK, B)

    zeros128 = jnp.zeros((NPAD, D), jnp.float32)
    zeros16 = jnp.zeros((NPAD, 16), jnp.float32)

    degp = _histones(dst_s, zeros16)
    g1, dinv16 = _tc1(x, W1, degp[:N], degp[NPAD:NPAD + N])

    cp = _segsum(dinv16, dst_g, src_s, zeros16, 16)

    s1p = _segsum(g1, src_g, dst_s, zeros128, D)
    g2 = _tc2(s1p[:N], s1p[NPAD:NPAD + N], g1, dinv16, b1.reshape(1, D), W2)

    s2p = _segsum(g2, src_g, dst_s, zeros128, D)
    out = _tc3(s2p[:N], s2p[NPAD:NPAD + N], g2, dinv16,
               cp[:N], cp[NPAD:NPAD + N],
               b2.reshape(1, D), W3, b3.reshape(1, D))
    return out[0]
